# baseline (speedup 1.0000x reference)
import functools

import jax
import jax.numpy as jnp
from jax import lax
from jax.experimental import pallas as pl
from jax.experimental.pallas import tpu as pltpu
from jax.experimental.pallas import tpu_sc as plsc

_DIM = 32
_SEQ = 50
_PAD = 128
_BT = 128
_NW = 32
_Q = 10


def _gather_body(table_hbm, idx_hbm, out_hbm, idx_vb, idx_c, g_all, stage,
                 gsem, osem):
    n_batch = idx_hbm.shape[0]
    nbt_w = n_batch // _BT // _NW
    wid = lax.axis_index("s") * 2 + lax.axis_index("c")
    lane = lax.iota(jnp.int32, 16)

    def unit(u, carry):
        btl = (u * 13) >> 6
        s0 = (u - btl * 5) * _Q
        bt = wid * nbt_w + btl
        b0 = pl.multiple_of(bt * _BT, _BT)

        @pl.when(s0 == 0)
        def _():
            pltpu.sync_copy(idx_hbm.at[pl.ds(b0, _BT)], idx_vb)

        def build(i, c_):
            for k in range(8):
                v = plsc.load_gather(
                    idx_vb, [k * 16 + lane, s0 + i + 0 * lane])
                idx_c[i, pl.ds(k * 16, 16)] = plsc.bitcast(v, jnp.int32)
            return c_

        lax.fori_loop(0, _Q, build, 0)

        def fire(i, c_):
            pltpu.async_copy(table_hbm.at[idx_c.at[i]], g_all.at[i], gsem)
            return c_

        lax.fori_loop(0, _Q, fire, 0)

        def drain_g(i, c_):
            pltpu.make_async_copy(
                table_hbm.at[idx_c.at[0]], g_all.at[0], gsem).wait()
            return c_

        lax.fori_loop(0, _Q, drain_g, 0)

        def xpose(i, c_):
            for c in range(_DIM):
                ct, ci = c // 8, c % 8
                for k in range(8):
                    v = plsc.load_gather(
                        g_all, [i + 0 * lane, k * 16 + lane, c + 0 * lane])
                    stage[i, ct, ci, pl.ds(k * 16, 16)] = v
            return c_

        lax.fori_loop(0, _Q, xpose, 0)

        def put(i, c_):
            for ct in range(4):
                pltpu.async_copy(
                    stage.at[i, ct], out_hbm.at[s0 + i, ct, bt], osem)
            return c_

        lax.fori_loop(0, _Q, put, 0)

        def drain_o(i, c_):
            for ct in range(4):
                pltpu.make_async_copy(
                    stage.at[0, 0], out_hbm.at[0, 0, 0], osem).wait()
            return c_

        lax.fori_loop(0, _Q, drain_o, 0)
        return carry

    lax.fori_loop(0, nbt_w * 5, unit, 0)


@functools.partial(jax.jit, static_argnums=2)
def _sc_gather(idx_f, weight, n_batch):
    mesh = plsc.VectorSubcoreMesh(core_axis_name="c", subcore_axis_name="s")
    return pl.kernel(
        _gather_body,
        out_type=jax.ShapeDtypeStruct(
            (_SEQ, _DIM // 8, n_batch // _BT, 8, _BT), jnp.float32),
        mesh=mesh,
        scratch_types=[
            pltpu.VMEM((_BT, _PAD), jnp.float32),
            pltpu.VMEM((_Q, _BT), jnp.int32),
            pltpu.VMEM((_Q, _BT, _DIM), jnp.float32),
            pltpu.VMEM((_Q, _DIM // 8, 8, _BT), jnp.float32),
            pltpu.SemaphoreType.DMA,
            pltpu.SemaphoreType.DMA,
        ],
        compiler_params=pltpu.CompilerParams(
            use_tc_tiling_on_sc=False, needs_layout_passes=False),
    )(weight, idx_f)


def kernel(indices, weight):
    n_batch = indices.shape[0]
    idx_pad = jnp.pad(indices.astype(jnp.int32), ((0, 0), (0, _PAD - _SEQ)))
    idx_f = jax.lax.bitcast_convert_type(idx_pad, jnp.float32)
    out_k = _sc_gather(idx_f, weight, n_batch)
    return out_k.transpose(2, 4, 0, 1, 3).reshape(n_batch, _SEQ, _DIM)


# speedup vs baseline: 1.0002x; 1.0002x over previous
import functools

import jax
import jax.numpy as jnp
from jax import lax
from jax.experimental import pallas as pl
from jax.experimental.pallas import tpu as pltpu
from jax.experimental.pallas import tpu_sc as plsc

_DIM = 32
_SEQ = 50
_PAD = 128
_BT = 128
_NW = 32
_Q = 10


def _gather_body(table_hbm, idx_hbm, out_hbm, idx_vb, idx_c, g_all, stage,
                 gsem, osem):
    n_batch = idx_hbm.shape[0]
    nbt_w = n_batch // _BT // _NW
    wid = lax.axis_index("s") * 2 + lax.axis_index("c")
    lane = lax.iota(jnp.int32, 16)

    def unit(u, carry):
        btl = (u * 13) >> 6
        s0 = (u - btl * 5) * _Q
        bt = wid * nbt_w + btl
        b0 = pl.multiple_of(bt * _BT, _BT)

        @pl.when(s0 == 0)
        def _():
            pltpu.sync_copy(idx_hbm.at[pl.ds(b0, _BT)], idx_vb)

        def build(i, c_):
            for k in range(8):
                v = plsc.load_gather(
                    idx_vb, [k * 16 + lane, s0 + i + 0 * lane])
                idx_c[i, pl.ds(k * 16, 16)] = plsc.bitcast(v, jnp.int32)
            return c_

        lax.fori_loop(0, _Q, build, 0)

        def fire(i, c_):
            pltpu.async_copy(table_hbm.at[idx_c.at[i]], g_all.at[i], gsem)
            return c_

        lax.fori_loop(0, _Q, fire, 0)

        def drain_g(i, c_):
            pltpu.make_async_copy(
                table_hbm.at[pl.ds(0, _BT)], g_all.at[0], gsem).wait()
            return c_

        lax.fori_loop(0, _Q, drain_g, 0)

        def xpose(i, c_):
            for c in range(_DIM):
                ct, ci = c // 8, c % 8
                for k in range(8):
                    v = plsc.load_gather(
                        g_all, [i + 0 * lane, k * 16 + lane, c + 0 * lane])
                    stage[i, ct, ci, pl.ds(k * 16, 16)] = v
            return c_

        lax.fori_loop(0, _Q, xpose, 0)

        pltpu.sync_copy(stage, out_hbm.at[pl.ds(s0, _Q), :, bt])
        return carry

    lax.fori_loop(0, nbt_w * 5, unit, 0)


@functools.partial(jax.jit, static_argnums=2)
def _sc_gather(idx_f, weight, n_batch):
    mesh = plsc.VectorSubcoreMesh(core_axis_name="c", subcore_axis_name="s")
    return pl.kernel(
        _gather_body,
        out_type=jax.ShapeDtypeStruct(
            (_SEQ, _DIM // 8, n_batch // _BT, 8, _BT), jnp.float32),
        mesh=mesh,
        scratch_types=[
            pltpu.VMEM((_BT, _PAD), jnp.float32),
            pltpu.VMEM((_Q, _BT), jnp.int32),
            pltpu.VMEM((_Q, _BT, _DIM), jnp.float32),
            pltpu.VMEM((_Q, _DIM // 8, 8, _BT), jnp.float32),
            pltpu.SemaphoreType.DMA,
            pltpu.SemaphoreType.DMA,
        ],
        compiler_params=pltpu.CompilerParams(
            use_tc_tiling_on_sc=False, needs_layout_passes=False),
    )(weight, idx_f)


def kernel(indices, weight):
    n_batch = indices.shape[0]
    idx_pad = jnp.pad(indices.astype(jnp.int32), ((0, 0), (0, _PAD - _SEQ)))
    idx_f = jax.lax.bitcast_convert_type(idx_pad, jnp.float32)
    out_k = _sc_gather(idx_f, weight, n_batch)
    return out_k.transpose(2, 4, 0, 1, 3).reshape(n_batch, _SEQ, _DIM)


# speedup vs baseline: 1.0219x; 1.0217x over previous
import functools

import jax
import jax.numpy as jnp
from jax import lax
from jax.experimental import pallas as pl
from jax.experimental.pallas import tpu as pltpu
from jax.experimental.pallas import tpu_sc as plsc

_DIM = 32
_SEQ = 50
_PAD = 128
_BT = 128
_NW = 32
_Q = 5


def _gather_body(table_hbm, idx_hbm, out_hbm, idx_vb, idx_c, g_all, stage,
                 gsa, gsb, osa, osb):
    n_batch = idx_hbm.shape[0]
    nbt_w = n_batch // _BT // _NW
    upb = _SEQ // _Q
    n_unit = nbt_w * upb
    wid = lax.axis_index("s") * 2 + lax.axis_index("c")
    lane = lax.iota(jnp.int32, 16)

    def unit_pos(u):
        btl = (u * 13) >> 7
        s0 = (u - btl * upb) * _Q
        bt = wid * nbt_w + btl
        return s0, bt

    def build(u, p):
        s0, bt = unit_pos(u)
        b0 = pl.multiple_of(bt * _BT, _BT)

        @pl.when(s0 == 0)
        def _():
            pltpu.sync_copy(idx_hbm.at[pl.ds(b0, _BT)], idx_vb)

        def body(i, c_):
            for k in range(8):
                v = plsc.load_gather(
                    idx_vb, [k * 16 + lane, s0 + i + 0 * lane])
                idx_c[p, i, pl.ds(k * 16, 16)] = plsc.bitcast(v, jnp.int32)
            return c_

        lax.fori_loop(0, _Q, body, 0)

    def fire(p, sem):
        def body(i, c_):
            pltpu.async_copy(
                table_hbm.at[idx_c.at[p, i]], g_all.at[p, i], sem)
            return c_

        lax.fori_loop(0, _Q, body, 0)

    def drain_gather(sem):
        def body(i, c_):
            pltpu.make_async_copy(
                table_hbm.at[pl.ds(0, _BT)], g_all.at[0, 0], sem).wait()
            return c_

        lax.fori_loop(0, _Q, body, 0)

    def xpose(p):
        def body(i, c_):
            for c in range(_DIM):
                ct, ci = c // 8, c % 8
                for k in range(8):
                    v = plsc.load_gather(
                        g_all, [p + 0 * lane, i + 0 * lane, k * 16 + lane,
                                c + 0 * lane])
                    stage[p, i, ct, ci, pl.ds(k * 16, 16)] = v
            return c_

        lax.fori_loop(0, _Q, body, 0)

    def put(u, p, sem):
        s0, bt = unit_pos(u)
        pltpu.async_copy(stage.at[p], out_hbm.at[pl.ds(s0, _Q), :, bt], sem)

    def drain_put(p, sem):
        pltpu.make_async_copy(
            stage.at[p], out_hbm.at[pl.ds(0, _Q), :, 0], sem).wait()

    build(0, 0)
    fire(0, gsa)

    def pair(t, carry):
        u0 = t * 2
        u1 = u0 + 1

        build(u1, 1)
        fire(1, gsb)
        drain_gather(gsa)

        @pl.when(t > 0)
        def _():
            drain_put(0, osa)

        xpose(0)
        put(u0, 0, osa)

        @pl.when(u0 + 2 < n_unit)
        def _():
            build(u0 + 2, 0)
            fire(0, gsa)

        drain_gather(gsb)

        @pl.when(t > 0)
        def _():
            drain_put(1, osb)

        xpose(1)
        put(u1, 1, osb)
        return carry

    lax.fori_loop(0, n_unit // 2, pair, 0)
    drain_put(0, osa)
    drain_put(1, osb)


@functools.partial(jax.jit, static_argnums=2)
def _sc_gather(idx_f, weight, n_batch):
    mesh = plsc.VectorSubcoreMesh(core_axis_name="c", subcore_axis_name="s")
    return pl.kernel(
        _gather_body,
        out_type=jax.ShapeDtypeStruct(
            (_SEQ, _DIM // 8, n_batch // _BT, 8, _BT), jnp.float32),
        mesh=mesh,
        scratch_types=[
            pltpu.VMEM((_BT, _PAD), jnp.float32),
            pltpu.VMEM((2, _Q, _BT), jnp.int32),
            pltpu.VMEM((2, _Q, _BT, _DIM), jnp.float32),
            pltpu.VMEM((2, _Q, _DIM // 8, 8, _BT), jnp.float32),
            pltpu.SemaphoreType.DMA,
            pltpu.SemaphoreType.DMA,
            pltpu.SemaphoreType.DMA,
            pltpu.SemaphoreType.DMA,
        ],
        compiler_params=pltpu.CompilerParams(
            use_tc_tiling_on_sc=False, needs_layout_passes=False),
    )(weight, idx_f)


def kernel(indices, weight):
    n_batch = indices.shape[0]
    idx_pad = jnp.pad(indices.astype(jnp.int32), ((0, 0), (0, _PAD - _SEQ)))
    idx_f = jax.lax.bitcast_convert_type(idx_pad, jnp.float32)
    out_k = _sc_gather(idx_f, weight, n_batch)
    return out_k.transpose(2, 4, 0, 1, 3).reshape(n_batch, _SEQ, _DIM)


# speedup vs baseline: 1.2021x; 1.1764x over previous
import functools

import jax
import jax.numpy as jnp
from jax import lax
from jax.experimental import pallas as pl
from jax.experimental.pallas import tpu as pltpu
from jax.experimental.pallas import tpu_sc as plsc

_DIM = 32
_SEQ = 50
_PAD = 128
_BT = 128
_NW = 32
_Q = 5


def _gather_body(table_hbm, idx_hbm, out_hbm, idx_vb, idx_c, g_all, stage,
                 gsa, gsb, osa, osb):
    n_batch = idx_hbm.shape[0]
    nbt_w = n_batch // _BT // _NW
    upb = _SEQ // _Q
    n_unit = nbt_w * upb
    wid = lax.axis_index("s") * 2 + lax.axis_index("c")
    lane = lax.iota(jnp.int32, 16)

    def unit_pos(u):
        btl = (u * 13) >> 7
        s0 = (u - btl * upb) * _Q
        bt = wid * nbt_w + btl
        return s0, bt

    def build(u, p):
        s0, bt = unit_pos(u)
        b0 = pl.multiple_of(bt * _BT, _BT)

        @pl.when(s0 == 0)
        def _():
            pltpu.sync_copy(idx_hbm.at[pl.ds(b0, _BT)], idx_vb)

        def body(i, c_):
            for k in range(8):
                v = plsc.load_gather(
                    idx_vb, [k * 16 + lane, s0 + i + 0 * lane])
                idx_c[p, i, pl.ds(k * 16, 16)] = plsc.bitcast(v, jnp.int32)
            return c_

        lax.fori_loop(0, _Q, body, 0)

    def fire(p, sem):
        def body(i, c_):
            pltpu.async_copy(
                table_hbm.at[idx_c.at[p, i]], g_all.at[p, i], sem)
            return c_

        lax.fori_loop(0, _Q, body, 0)

    def drain_gather(sem):
        def body(i, c_):
            pltpu.make_async_copy(
                table_hbm.at[pl.ds(0, _BT)], g_all.at[0, 0], sem).wait()
            return c_

        lax.fori_loop(0, _Q, body, 0)

    ct_lo = lane >> 3
    ci_v = lane & 7
    ct_hi = ct_lo + 2

    def xpose(p):
        stage_p = stage.at[p]

        def body(r, c_):
            i = r >> 7
            bi = r - (i << 7)
            v1 = g_all[p, i, bi, pl.ds(0, 16)]
            v2 = g_all[p, i, bi, pl.ds(16, 16)]
            iv = i + 0 * lane
            bv = bi + 0 * lane
            plsc.store_scatter(stage_p, [iv, ct_lo, ci_v, bv], v1)
            plsc.store_scatter(stage_p, [iv, ct_hi, ci_v, bv], v2)
            return c_

        lax.fori_loop(0, _Q * _BT, body, 0)

    def put(u, p, sem):
        s0, bt = unit_pos(u)
        pltpu.async_copy(stage.at[p], out_hbm.at[pl.ds(s0, _Q), :, bt], sem)

    def drain_put(p, sem):
        pltpu.make_async_copy(
            stage.at[p], out_hbm.at[pl.ds(0, _Q), :, 0], sem).wait()

    build(0, 0)
    fire(0, gsa)

    def pair(t, carry):
        u0 = t * 2
        u1 = u0 + 1

        build(u1, 1)
        fire(1, gsb)
        drain_gather(gsa)

        @pl.when(t > 0)
        def _():
            drain_put(0, osa)

        xpose(0)
        put(u0, 0, osa)

        @pl.when(u0 + 2 < n_unit)
        def _():
            build(u0 + 2, 0)
            fire(0, gsa)

        drain_gather(gsb)

        @pl.when(t > 0)
        def _():
            drain_put(1, osb)

        xpose(1)
        put(u1, 1, osb)
        return carry

    lax.fori_loop(0, n_unit // 2, pair, 0)
    drain_put(0, osa)
    drain_put(1, osb)


@functools.partial(jax.jit, static_argnums=2)
def _sc_gather(idx_f, weight, n_batch):
    mesh = plsc.VectorSubcoreMesh(core_axis_name="c", subcore_axis_name="s")
    return pl.kernel(
        _gather_body,
        out_type=jax.ShapeDtypeStruct(
            (_SEQ, _DIM // 8, n_batch // _BT, 8, _BT), jnp.float32),
        mesh=mesh,
        scratch_types=[
            pltpu.VMEM((_BT, _PAD), jnp.float32),
            pltpu.VMEM((2, _Q, _BT), jnp.int32),
            pltpu.VMEM((2, _Q, _BT, _DIM), jnp.float32),
            pltpu.VMEM((2, _Q, _DIM // 8, 8, _BT), jnp.float32),
            pltpu.SemaphoreType.DMA,
            pltpu.SemaphoreType.DMA,
            pltpu.SemaphoreType.DMA,
            pltpu.SemaphoreType.DMA,
        ],
        compiler_params=pltpu.CompilerParams(
            use_tc_tiling_on_sc=False, needs_layout_passes=False),
    )(weight, idx_f)


def kernel(indices, weight):
    n_batch = indices.shape[0]
    idx_pad = jnp.pad(indices.astype(jnp.int32), ((0, 0), (0, _PAD - _SEQ)))
    idx_f = jax.lax.bitcast_convert_type(idx_pad, jnp.float32)
    out_k = _sc_gather(idx_f, weight, n_batch)
    return out_k.transpose(2, 4, 0, 1, 3).reshape(n_batch, _SEQ, _DIM)


# speedup vs baseline: 1.2024x; 1.0002x over previous
import functools

import jax
import jax.numpy as jnp
from jax import lax
from jax.experimental import pallas as pl
from jax.experimental.pallas import tpu as pltpu
from jax.experimental.pallas import tpu_sc as plsc

_DIM = 32
_SEQ = 50
_PAD = 128
_BT = 128
_NW = 32
_Q = 5


def _gather_body(table_hbm, idx_hbm, out_hbm, idx_vb, idx_c, g_all, stage,
                 gsa, gsb, osa, osb):
    n_batch = idx_hbm.shape[0]
    nbt_w = n_batch // _BT // _NW
    upb = _SEQ // _Q
    n_unit = nbt_w * upb
    wid = lax.axis_index("s") * 2 + lax.axis_index("c")
    lane = lax.iota(jnp.int32, 16)

    def unit_pos(u):
        btl = (u * 13) >> 7
        s0 = (u - btl * upb) * _Q
        bt = wid * nbt_w + btl
        return s0, bt

    def build(u, p):
        s0, bt = unit_pos(u)
        b0 = pl.multiple_of(bt * _BT, _BT)

        @pl.when(s0 == 0)
        def _():
            pltpu.sync_copy(idx_hbm.at[pl.ds(b0, _BT)], idx_vb)

        def body(i, c_):
            for k in range(8):
                v = plsc.load_gather(
                    idx_vb, [k * 16 + lane, s0 + i + 0 * lane])
                idx_c[p, i, pl.ds(k * 16, 16)] = plsc.bitcast(v, jnp.int32)
            return c_

        lax.fori_loop(0, _Q, body, 0)

    def fire(p, sem):
        def body(i, c_):
            pltpu.async_copy(
                table_hbm.at[idx_c.at[p, i]], g_all.at[p, i], sem)
            return c_

        lax.fori_loop(0, _Q, body, 0)

    def drain_gather(sem):
        def body(i, c_):
            pltpu.make_async_copy(
                table_hbm.at[pl.ds(0, _BT)], g_all.at[0, 0], sem).wait()
            return c_

        lax.fori_loop(0, _Q, body, 0)

    ct_lo = lane >> 3
    ct_hi = ct_lo + 2
    inner = (lane & 7) * _BT

    def xpose(p):
        stage_p = stage.at[p]

        def body(r, c_):
            i = r >> 7
            bi = r - (i << 7)
            v1 = g_all[p, i, bi, pl.ds(0, 16)]
            v2 = g_all[p, i, bi, pl.ds(16, 16)]
            iv = i + 0 * lane
            bv = inner + bi
            plsc.store_scatter(stage_p, [iv, ct_lo, bv], v1)
            plsc.store_scatter(stage_p, [iv, ct_hi, bv], v2)
            return c_

        lax.fori_loop(0, _Q * _BT, body, 0)

    def put(u, p, sem):
        s0, bt = unit_pos(u)
        pltpu.async_copy(stage.at[p], out_hbm.at[pl.ds(s0, _Q), :, bt], sem)

    def drain_put(p, sem):
        pltpu.make_async_copy(
            stage.at[p], out_hbm.at[pl.ds(0, _Q), :, 0], sem).wait()

    build(0, 0)
    fire(0, gsa)

    def pair(t, carry):
        u0 = t * 2
        u1 = u0 + 1

        build(u1, 1)
        fire(1, gsb)
        drain_gather(gsa)

        @pl.when(t > 0)
        def _():
            drain_put(0, osa)

        xpose(0)
        put(u0, 0, osa)

        @pl.when(u0 + 2 < n_unit)
        def _():
            build(u0 + 2, 0)
            fire(0, gsa)

        drain_gather(gsb)

        @pl.when(t > 0)
        def _():
            drain_put(1, osb)

        xpose(1)
        put(u1, 1, osb)
        return carry

    lax.fori_loop(0, n_unit // 2, pair, 0)
    drain_put(0, osa)
    drain_put(1, osb)


@functools.partial(jax.jit, static_argnums=2)
def _sc_gather(idx_f, weight, n_batch):
    mesh = plsc.VectorSubcoreMesh(core_axis_name="c", subcore_axis_name="s")
    return pl.kernel(
        _gather_body,
        out_type=jax.ShapeDtypeStruct(
            (_SEQ, _DIM // 8, n_batch // _BT, 8 * _BT), jnp.float32),
        mesh=mesh,
        scratch_types=[
            pltpu.VMEM((_BT, _PAD), jnp.float32),
            pltpu.VMEM((2, _Q, _BT), jnp.int32),
            pltpu.VMEM((2, _Q, _BT, _DIM), jnp.float32),
            pltpu.VMEM((2, _Q, _DIM // 8, 8 * _BT), jnp.float32),
            pltpu.SemaphoreType.DMA,
            pltpu.SemaphoreType.DMA,
            pltpu.SemaphoreType.DMA,
            pltpu.SemaphoreType.DMA,
        ],
        compiler_params=pltpu.CompilerParams(
            use_tc_tiling_on_sc=False, needs_layout_passes=False),
    )(weight, idx_f)


def kernel(indices, weight):
    n_batch = indices.shape[0]
    idx_pad = jnp.pad(indices.astype(jnp.int32), ((0, 0), (0, _PAD - _SEQ)))
    idx_f = jax.lax.bitcast_convert_type(idx_pad, jnp.float32)
    out_k = _sc_gather(idx_f, weight, n_batch)
    out_k = out_k.reshape(_SEQ, _DIM // 8, n_batch // _BT, 8, _BT)
    return out_k.transpose(2, 4, 0, 1, 3).reshape(n_batch, _SEQ, _DIM)
